# baseline (speedup 1.0000x reference)
import functools

import jax
import jax.numpy as jnp
from jax import lax
from jax.experimental import pallas as pl
from jax.experimental.pallas import tpu as pltpu
from jax.experimental.pallas import tpu_sc as plsc

B, S, D = 4, 1024, 768
N_FLAT = B * S
LANES = 16
CHUNK = 64
D_VECS = D // LANES


def _make_kernel():
    info = plsc.get_sparse_core_info()
    nc, ns = info.num_cores, info.num_subcores
    nw = nc * ns
    per_w = N_FLAT // nw
    n_chunks = per_w // CHUNK

    mesh = plsc.VectorSubcoreMesh(core_axis_name="c", subcore_axis_name="s")

    @functools.partial(
        pl.kernel,
        mesh=mesh,
        out_type=jax.ShapeDtypeStruct((N_FLAT, D), jnp.float32),
        scratch_types=[
            pltpu.VMEM((CHUNK,), jnp.int32),
            pltpu.VMEM((CHUNK, D), jnp.float32),
            pltpu.VMEM((CHUNK, D), jnp.float32),
            pltpu.SemaphoreType.DMA,
            pltpu.SemaphoreType.DMA,
        ],
    )
    def k(ids_hbm, wte_hbm, wpe_hbm, out_hbm, idx_v, rows_v, wpe_v, sem_g, sem_w):
        wid = lax.axis_index("s") * nc + lax.axis_index("c")
        base = wid * per_w
        pos_base = lax.rem(base, S)

        for c in range(n_chunks):
            start = base + c * CHUNK
            pstart = pos_base + c * CHUNK
            pltpu.sync_copy(ids_hbm.at[pl.ds(start, CHUNK)], idx_v)
            gcp = pltpu.async_copy(wte_hbm.at[idx_v], rows_v, sem_g)
            wcp = pltpu.async_copy(wpe_hbm.at[pl.ds(pstart, CHUNK)], wpe_v, sem_w)
            gcp.wait()
            wcp.wait()

            def body(r, carry):
                for d in range(D_VECS):
                    x = wpe_v[r, pl.ds(d * LANES, LANES)]
                    plsc.addupdate(rows_v.at[r, pl.ds(d * LANES, LANES)], x)
                return carry

            lax.fori_loop(0, CHUNK, body, 0)
            pltpu.sync_copy(rows_v, out_hbm.at[pl.ds(start, CHUNK)])

    return k


_sc_kernel = _make_kernel()


@jax.jit
def kernel(input_ids, wte, wpe):
    ids = input_ids.reshape(-1).astype(jnp.int32)
    out = _sc_kernel(ids, wte, wpe)
    return out.reshape(B, S, D)


# speedup vs baseline: 1.0584x; 1.0584x over previous
import functools

import jax
import jax.numpy as jnp
from jax import lax
from jax.experimental import pallas as pl
from jax.experimental.pallas import tpu as pltpu
from jax.experimental.pallas import tpu_sc as plsc

B, S, D = 4, 1024, 768
N_FLAT = B * S
LANES = 16
CHUNK = 32
D_VECS = D // LANES


def _make_kernel():
    info = plsc.get_sparse_core_info()
    nc, ns = info.num_cores, info.num_subcores
    nw = nc * ns
    per_w = N_FLAT // nw
    n_chunks = per_w // CHUNK

    mesh = plsc.VectorSubcoreMesh(core_axis_name="c", subcore_axis_name="s")

    @functools.partial(
        pl.kernel,
        mesh=mesh,
        out_type=jax.ShapeDtypeStruct((N_FLAT, D), jnp.float32),
        scratch_types=[
            pltpu.VMEM((per_w,), jnp.int32),
            pltpu.VMEM((CHUNK, D), jnp.float32),
            pltpu.VMEM((CHUNK, D), jnp.float32),
            pltpu.VMEM((CHUNK, D), jnp.float32),
            pltpu.VMEM((CHUNK, D), jnp.float32),
            pltpu.SemaphoreType.DMA,
            pltpu.SemaphoreType.DMA,
            pltpu.SemaphoreType.DMA,
            pltpu.SemaphoreType.DMA,
            pltpu.SemaphoreType.DMA,
            pltpu.SemaphoreType.DMA,
        ],
    )
    def k(ids_hbm, wte_hbm, wpe_hbm, out_hbm, idx_v,
          rows0, rows1, wpe0, wpe1, sg0, sg1, sw0, sw1, so0, so1):
        wid = lax.axis_index("s") * nc + lax.axis_index("c")
        base = wid * per_w
        pos_base = lax.rem(base, S)

        rows = (rows0, rows1)
        wpes = (wpe0, wpe1)
        sgs = (sg0, sg1)
        sws = (sw0, sw1)
        sos = (so0, so1)

        pltpu.sync_copy(ids_hbm.at[pl.ds(base, per_w)], idx_v)

        def start(c):
            b = c & 1
            g = pltpu.async_copy(
                wte_hbm.at[idx_v.at[pl.ds(c * CHUNK, CHUNK)]], rows[b], sgs[b])
            w = pltpu.async_copy(
                wpe_hbm.at[pl.ds(pos_base + c * CHUNK, CHUNK)], wpes[b], sws[b])
            return g, w

        pending = start(0)
        out_pending = [None, None]
        for c in range(n_chunks):
            b = c & 1
            nb = (c + 1) & 1
            if c + 1 < n_chunks:
                if out_pending[nb] is not None:
                    out_pending[nb].wait()
                    out_pending[nb] = None
                nxt = start(c + 1)
            g, w = pending
            g.wait()
            w.wait()

            def body(r, carry):
                for d in range(D_VECS):
                    x = wpes[b][r, pl.ds(d * LANES, LANES)]
                    plsc.addupdate(rows[b].at[r, pl.ds(d * LANES, LANES)], x)
                return carry

            lax.fori_loop(0, CHUNK, body, 0)
            out_pending[b] = pltpu.async_copy(
                rows[b], out_hbm.at[pl.ds(base + c * CHUNK, CHUNK)], sos[b])
            if c + 1 < n_chunks:
                pending = nxt
        for op in out_pending:
            if op is not None:
                op.wait()

    return k


_sc_kernel = _make_kernel()


@jax.jit
def kernel(input_ids, wte, wpe):
    ids = input_ids.reshape(-1).astype(jnp.int32)
    out = _sc_kernel(ids, wte, wpe)
    return out.reshape(B, S, D)


# speedup vs baseline: 1.1697x; 1.1051x over previous
import functools

import jax
import jax.numpy as jnp
from jax import lax
from jax.experimental import pallas as pl
from jax.experimental.pallas import tpu as pltpu
from jax.experimental.pallas import tpu_sc as plsc

B, S, D = 4, 1024, 768
LANES = 16
D_VECS = D // LANES


def _make_kernel():
    info = plsc.get_sparse_core_info()
    nc, ns = info.num_cores, info.num_subcores
    nw = nc * ns
    pos_w = S // nw

    mesh = plsc.VectorSubcoreMesh(core_axis_name="c", subcore_axis_name="s")

    @functools.partial(
        pl.kernel,
        mesh=mesh,
        out_type=jax.ShapeDtypeStruct((B, S, D), jnp.float32),
        scratch_types=[
            pltpu.VMEM((B, pos_w), jnp.int32),
            pltpu.VMEM((pos_w, D), jnp.float32),
            pltpu.VMEM((pos_w, D), jnp.float32),
            pltpu.VMEM((pos_w, D), jnp.float32),
            pltpu.SemaphoreType.DMA,
            pltpu.SemaphoreType.DMA,
            pltpu.SemaphoreType.DMA,
            pltpu.SemaphoreType.DMA,
            pltpu.SemaphoreType.DMA,
        ],
    )
    def k(ids_hbm, wte_hbm, wpe_hbm, out_hbm, idx_v, wpe_v,
          rows0, rows1, sg0, sg1, so0, so1, sw):
        wid = lax.axis_index("s") * nc + lax.axis_index("c")
        col = wid * pos_w

        rows = (rows0, rows1)
        sgs = (sg0, sg1)
        sos = (so0, so1)

        wcp = pltpu.async_copy(wpe_hbm.at[pl.ds(col, pos_w)], wpe_v, sw)
        for b in range(B):
            pltpu.sync_copy(ids_hbm.at[b, pl.ds(col, pos_w)], idx_v.at[b])

        def start(b):
            return pltpu.async_copy(
                wte_hbm.at[idx_v.at[b]], rows[b & 1], sgs[b & 1])

        pending = start(0)
        out_pending = [None, None]
        for b in range(B):
            if b + 1 < B:
                if out_pending[(b + 1) & 1] is not None:
                    out_pending[(b + 1) & 1].wait()
                    out_pending[(b + 1) & 1] = None
                nxt = start(b + 1)
            pending.wait()
            if b == 0:
                wcp.wait()

            def body(r, carry):
                for d in range(D_VECS):
                    x = wpe_v[r, pl.ds(d * LANES, LANES)]
                    plsc.addupdate(rows[b & 1].at[r, pl.ds(d * LANES, LANES)], x)
                return carry

            lax.fori_loop(0, pos_w, body, 0)
            out_pending[b & 1] = pltpu.async_copy(
                rows[b & 1], out_hbm.at[b, pl.ds(col, pos_w)], sos[b & 1])
            if b + 1 < B:
                pending = nxt
        for op in out_pending:
            if op is not None:
                op.wait()

    return k


_sc_kernel = _make_kernel()


@jax.jit
def kernel(input_ids, wte, wpe):
    ids = input_ids
    if ids.dtype != jnp.int32:
        ids = ids.astype(jnp.int32)
    return _sc_kernel(ids, wte, wpe)


# speedup vs baseline: 1.2313x; 1.0527x over previous
import functools

import jax
import jax.numpy as jnp
from jax import lax
from jax.experimental import pallas as pl
from jax.experimental.pallas import tpu as pltpu
from jax.experimental.pallas import tpu_sc as plsc

B, S, D = 4, 1024, 768
LANES = 16
D_VECS = D // LANES
UNROLL = 16


def _make_kernel():
    info = plsc.get_sparse_core_info()
    nc, ns = info.num_cores, info.num_subcores
    nw = nc * ns
    pos_w = S // nw

    mesh = plsc.VectorSubcoreMesh(core_axis_name="c", subcore_axis_name="s")

    @functools.partial(
        pl.kernel,
        mesh=mesh,
        out_type=jax.ShapeDtypeStruct((B, S, D), jnp.float32),
        scratch_types=[
            pltpu.VMEM((B, pos_w), jnp.int32),
            pltpu.VMEM((pos_w, D), jnp.float32),
            pltpu.VMEM((pos_w, D), jnp.float32),
            pltpu.VMEM((pos_w, D), jnp.float32),
            pltpu.VMEM((pos_w, D), jnp.float32),
            pltpu.VMEM((pos_w, D), jnp.float32),
            pltpu.SemaphoreType.DMA,
            pltpu.SemaphoreType.DMA,
            pltpu.SemaphoreType.DMA,
            pltpu.SemaphoreType.DMA,
            pltpu.SemaphoreType.DMA,
            pltpu.SemaphoreType.DMA,
            pltpu.SemaphoreType.DMA,
            pltpu.SemaphoreType.DMA,
            pltpu.SemaphoreType.DMA,
        ],
    )
    def k(ids_hbm, wte_hbm, wpe_hbm, out_hbm, idx_v, wpe_v,
          rows0, rows1, rows2, rows3,
          sg0, sg1, sg2, sg3, so0, so1, so2, so3, sw):
        wid = lax.axis_index("s") * nc + lax.axis_index("c")
        col = wid * pos_w

        rows = (rows0, rows1, rows2, rows3)
        sgs = (sg0, sg1, sg2, sg3)
        sos = (so0, so1, so2, so3)

        wcp = pltpu.async_copy(wpe_hbm.at[pl.ds(col, pos_w)], wpe_v, sw)
        for b in range(B):
            pltpu.sync_copy(ids_hbm.at[b, pl.ds(col, pos_w)], idx_v.at[b])
        gcps = [
            pltpu.async_copy(wte_hbm.at[idx_v.at[b]], rows[b], sgs[b])
            for b in range(B)
        ]

        out_cps = []
        for b in range(B):
            gcps[b].wait()
            if b == 0:
                wcp.wait()

            def body(i, carry):
                r = i // (D_VECS // UNROLL)
                d0 = (i % (D_VECS // UNROLL)) * UNROLL
                for u in range(UNROLL):
                    x = wpe_v[r, pl.ds(d0 * LANES + u * LANES, LANES)]
                    plsc.addupdate(
                        rows[b].at[r, pl.ds(d0 * LANES + u * LANES, LANES)], x)
                return carry

            lax.fori_loop(0, pos_w * (D_VECS // UNROLL), body, 0)
            out_cps.append(pltpu.async_copy(
                rows[b], out_hbm.at[b, pl.ds(col, pos_w)], sos[b]))
        for cp in out_cps:
            cp.wait()

    return k


_sc_kernel = _make_kernel()


@jax.jit
def kernel(input_ids, wte, wpe):
    ids = input_ids
    if ids.dtype != jnp.int32:
        ids = ids.astype(jnp.int32)
    return _sc_kernel(ids, wte, wpe)


# speedup vs baseline: 1.2746x; 1.0352x over previous
import functools

import jax
import jax.numpy as jnp
from jax import lax
from jax.experimental import pallas as pl
from jax.experimental.pallas import tpu as pltpu
from jax.experimental.pallas import tpu_sc as plsc

B, S, D = 4, 1024, 768
LANES = 16
D_VECS = D // LANES
UNROLL = 8


def _make_kernel():
    info = plsc.get_sparse_core_info()
    nc, ns = info.num_cores, info.num_subcores
    nw = nc * ns
    pos_w = S // nw

    mesh = plsc.VectorSubcoreMesh(core_axis_name="c", subcore_axis_name="s")

    @functools.partial(
        pl.kernel,
        mesh=mesh,
        out_type=jax.ShapeDtypeStruct((B, S, D), jnp.float32),
        scratch_types=[
            pltpu.VMEM((B, pos_w), jnp.int32),
            pltpu.VMEM((pos_w, D), jnp.float32),
            pltpu.VMEM((pos_w, D), jnp.float32),
            pltpu.VMEM((pos_w, D), jnp.float32),
            pltpu.VMEM((pos_w, D), jnp.float32),
            pltpu.VMEM((pos_w, D), jnp.float32),
            pltpu.SemaphoreType.DMA,
            pltpu.SemaphoreType.DMA,
            pltpu.SemaphoreType.DMA,
            pltpu.SemaphoreType.DMA,
            pltpu.SemaphoreType.DMA,
            pltpu.SemaphoreType.DMA,
            pltpu.SemaphoreType.DMA,
            pltpu.SemaphoreType.DMA,
            pltpu.SemaphoreType.DMA,
            pltpu.SemaphoreType.DMA,
            pltpu.SemaphoreType.DMA,
            pltpu.SemaphoreType.DMA,
            pltpu.SemaphoreType.DMA,
        ],
    )
    def k(ids_hbm, wte_hbm, wpe_hbm, out_hbm, idx_v, wpe_v,
          rows0, rows1, rows2, rows3,
          sg0, sg1, sg2, sg3, so0, so1, so2, so3,
          si0, si1, si2, si3, sw):
        wid = lax.axis_index("s") * nc + lax.axis_index("c")
        col = wid * pos_w

        rows = (rows0, rows1, rows2, rows3)
        sgs = (sg0, sg1, sg2, sg3)
        sos = (so0, so1, so2, so3)
        sis = (si0, si1, si2, si3)

        icps = [
            pltpu.async_copy(
                ids_hbm.at[b, pl.ds(col, pos_w)], idx_v.at[b], sis[b])
            for b in range(B)
        ]
        gcps = []
        wcp = None
        for b in range(B):
            icps[b].wait()
            gcps.append(
                pltpu.async_copy(wte_hbm.at[idx_v.at[b]], rows[b], sgs[b]))
            if b == 0:
                wcp = pltpu.async_copy(wpe_hbm.at[pl.ds(col, pos_w)], wpe_v, sw)

        out_cps = []
        for b in range(B):
            gcps[b].wait()
            if b == 0:
                wcp.wait()

            def body(i, carry):
                r = i // (D_VECS // UNROLL)
                d0 = (i % (D_VECS // UNROLL)) * UNROLL
                for u in range(UNROLL):
                    x = wpe_v[r, pl.ds(d0 * LANES + u * LANES, LANES)]
                    plsc.addupdate(
                        rows[b].at[r, pl.ds(d0 * LANES + u * LANES, LANES)], x)
                return carry

            lax.fori_loop(0, pos_w * (D_VECS // UNROLL), body, 0)
            out_cps.append(pltpu.async_copy(
                rows[b], out_hbm.at[b, pl.ds(col, pos_w)], sos[b]))
        for cp in out_cps:
            cp.wait()

    return k


_sc_kernel = _make_kernel()


@jax.jit
def kernel(input_ids, wte, wpe):
    ids = input_ids
    if ids.dtype != jnp.int32:
        ids = ids.astype(jnp.int32)
    return _sc_kernel(ids, wte, wpe)


# speedup vs baseline: 1.2762x; 1.0012x over previous
import functools

import jax
import jax.numpy as jnp
from jax import lax
from jax.experimental import pallas as pl
from jax.experimental.pallas import tpu as pltpu
from jax.experimental.pallas import tpu_sc as plsc

B, S, D = 4, 1024, 768
LANES = 16
D_VECS = D // LANES
UNROLL = 8


def _make_kernel():
    info = plsc.get_sparse_core_info()
    nc, ns = info.num_cores, info.num_subcores
    nw = nc * ns
    pos_w = S // nw

    mesh = plsc.VectorSubcoreMesh(core_axis_name="c", subcore_axis_name="s")

    @functools.partial(
        pl.kernel,
        mesh=mesh,
        out_type=jax.ShapeDtypeStruct((B, S, D), jnp.float32),
        scratch_types=[
            pltpu.VMEM((B, pos_w), jnp.int32),
            pltpu.VMEM((pos_w, D), jnp.float32),
            pltpu.VMEM((pos_w, D), jnp.float32),
            pltpu.VMEM((pos_w, D), jnp.float32),
            pltpu.VMEM((pos_w, D), jnp.float32),
            pltpu.VMEM((pos_w, D), jnp.float32),
            pltpu.SemaphoreType.DMA,
            pltpu.SemaphoreType.DMA,
            pltpu.SemaphoreType.DMA,
            pltpu.SemaphoreType.DMA,
            pltpu.SemaphoreType.DMA,
            pltpu.SemaphoreType.DMA,
            pltpu.SemaphoreType.DMA,
            pltpu.SemaphoreType.DMA,
            pltpu.SemaphoreType.DMA,
            pltpu.SemaphoreType.DMA,
            pltpu.SemaphoreType.DMA,
            pltpu.SemaphoreType.DMA,
            pltpu.SemaphoreType.DMA,
        ],
    )
    def k(ids_hbm, wte_hbm, wpe_hbm, out_hbm, idx_v, wpe_v,
          rows0, rows1, rows2, rows3,
          sg0, sg1, sg2, sg3, so0, so1, so2, so3,
          si0, si1, si2, si3, sw):
        wid = lax.axis_index("s") * nc + lax.axis_index("c")
        col = wid * pos_w

        rows = (rows0, rows1, rows2, rows3)
        sgs = (sg0, sg1, sg2, sg3)
        sos = (so0, so1, so2, so3)
        sis = (si0, si1, si2, si3)

        icps = [
            pltpu.async_copy(
                ids_hbm.at[b, pl.ds(col, pos_w)], idx_v.at[b], sis[b])
            for b in range(B)
        ]

        def start(b):
            icps[b].wait()
            return pltpu.async_copy(wte_hbm.at[idx_v.at[b]], rows[b], sgs[b])

        gcps = [start(0)]
        wcp = pltpu.async_copy(wpe_hbm.at[pl.ds(col, pos_w)], wpe_v, sw)
        gcps.append(start(1))

        out_cps = []
        for b in range(B):
            gcps[b].wait()
            if b == 0:
                wcp.wait()

            def body(i, carry):
                r = i // (D_VECS // UNROLL)
                d0 = (i % (D_VECS // UNROLL)) * UNROLL
                for u in range(UNROLL):
                    x = wpe_v[r, pl.ds(d0 * LANES + u * LANES, LANES)]
                    plsc.addupdate(
                        rows[b].at[r, pl.ds(d0 * LANES + u * LANES, LANES)], x)
                return carry

            lax.fori_loop(0, pos_w * (D_VECS // UNROLL), body, 0)
            out_cps.append(pltpu.async_copy(
                rows[b], out_hbm.at[b, pl.ds(col, pos_w)], sos[b]))
            if b + 2 < B:
                gcps.append(start(b + 2))
        for cp in out_cps:
            cp.wait()

    return k


_sc_kernel = _make_kernel()


@jax.jit
def kernel(input_ids, wte, wpe):
    ids = input_ids
    if ids.dtype != jnp.int32:
        ids = ids.astype(jnp.int32)
    return _sc_kernel(ids, wte, wpe)
